# baseline (speedup 1.0000x reference)
import functools

import jax
import jax.numpy as jnp
from jax import lax
from jax.experimental import pallas as pl
from jax.experimental.pallas import tpu as pltpu
from jax.experimental.pallas import tpu_sc as plsc

MARGIN = 12.0
B = 4096
C = 64
X, Y, Z = 100, 88, 80
XYZ = X * Y * Z
CP = 128

NC, NS, L = 2, 16, 16
NW = NC * NS
PPW = B // NW
NCHUNK = PPW // L


def _pmod(a, n):
  r = lax.rem(a, n)
  return jnp.where(r < 0, r + n, r)


def _voxel_chunk(xv, yv, zv, j):
  s = pl.ds(j * L, L)
  xx = _pmod(xv[s] - 25, X)
  yy = _pmod(yv[s] - 225, Y)
  zz = _pmod(zv[s] - 28, Z)
  return (xx * Y + yy) * Z + zz


def _make_sc_kernel():
  mesh = plsc.VectorSubcoreMesh(core_axis_name="c", subcore_axis_name="s")

  @functools.partial(
      pl.kernel,
      mesh=mesh,
      out_type=(
          jax.ShapeDtypeStruct((B * L,), jnp.float32),
          jax.ShapeDtypeStruct((B * L,), jnp.float32),
      ),
      scratch_types=[
          pltpu.VMEM((PPW,), jnp.int32),
          pltpu.VMEM((PPW,), jnp.int32),
          pltpu.VMEM((PPW,), jnp.int32),
          pltpu.VMEM((PPW,), jnp.int32),
          pltpu.VMEM((PPW,), jnp.int32),
          pltpu.VMEM((PPW,), jnp.int32),
          pltpu.VMEM((PPW * L,), jnp.float32),
          pltpu.VMEM((PPW * L,), jnp.float32),
          pltpu.VMEM((PPW * L,), jnp.float32),
          pltpu.VMEM((PPW, CP), jnp.float32),
          pltpu.VMEM((PPW, CP), jnp.float32),
          pltpu.VMEM((PPW, CP), jnp.float32),
          pltpu.VMEM((PPW * L,), jnp.float32),
          pltpu.VMEM((PPW * L,), jnp.float32),
          pltpu.SemaphoreType.DMA,
      ],
  )
  def sc_kernel(fx_hbm, fy_hbm, fz_hbm, px_hbm, py_hbm, pz_hbm,
                nx_hbm, ny_hbm, nz_hbm, hf_hbm, hp_hbm, hn_hbm,
                f_hbm, dp_hbm, dn_hbm,
                xv, yv, zv, idxf_v, idxp_v, idxn_v,
                hf_v, hp_v, hn_v, rvf, rvp, rvn, ov, ov2, sem):
    wid = lax.axis_index("s") * NC + lax.axis_index("c")
    base = wid * PPW

    pltpu.sync_copy(hf_hbm.at[pl.ds(base * L, PPW * L)], hf_v)
    pltpu.sync_copy(hp_hbm.at[pl.ds(base * L, PPW * L)], hp_v)
    pltpu.sync_copy(hn_hbm.at[pl.ds(base * L, PPW * L)], hn_v)

    for cx, cy, cz, idx_v, off in ((fx_hbm, fy_hbm, fz_hbm, idxf_v, 0),
                                   (px_hbm, py_hbm, pz_hbm, idxp_v, XYZ // 2),
                                   (nx_hbm, ny_hbm, nz_hbm, idxn_v, XYZ // 2)):
      pltpu.sync_copy(cx.at[pl.ds(base, PPW)], xv)
      pltpu.sync_copy(cy.at[pl.ds(base, PPW)], yv)
      pltpu.sync_copy(cz.at[pl.ds(base, PPW)], zv)
      for j in range(NCHUNK):
        idx_v[pl.ds(j * L, L)] = (lax.shift_right_logical(
            _voxel_chunk(xv, yv, zv, j), 1) + off)

    c1 = pltpu.async_copy(f_hbm.at[idxf_v], rvf, sem)
    c2 = pltpu.async_copy(f_hbm.at[idxp_v], rvp, sem)
    c3 = pltpu.async_copy(f_hbm.at[idxn_v], rvn, sem)
    c1.wait()
    c2.wait()
    c3.wait()

    def it(j, carry):
      for p in range(L):
        row = j * L + p
        hs = pl.ds(row * L, L)
        hf = hf_v[hs]
        hp = hp_v[hs]
        hn = hn_v[hs]
        accp = jnp.zeros((L,), jnp.float32)
        accn = jnp.zeros((L,), jnp.float32)
        for k in range(C // L):
          slo = pl.ds(k * L, L)
          shi = pl.ds(C + k * L, L)
          vf = rvf[row, slo]
          vf = vf + (rvf[row, shi] - vf) * hf
          vp = rvp[row, slo]
          vp = vp + (rvp[row, shi] - vp) * hp
          vn = rvn[row, slo]
          vn = vn + (rvn[row, shi] - vn) * hn
          accp = accp + (vf - vp) * (vf - vp)
          accn = accn + (vf - vn) * (vf - vn)
        ov[pl.ds(row * L, L)] = accp
        ov2[pl.ds(row * L, L)] = accn
      return carry

    lax.fori_loop(0, NCHUNK, it, 0)
    pltpu.sync_copy(ov, dp_hbm.at[pl.ds(base * L, PPW * L)])
    pltpu.sync_copy(ov2, dn_hbm.at[pl.ds(base * L, PPW * L)])

  return sc_kernel


_sc_gather_dist = _make_sc_kernel()


def _loss_tc(dp_ref, dn_ref, out_ref):
  dp = dp_ref[...]
  dn = dn_ref[...]
  sel = (lax.broadcasted_iota(jnp.int32, (128, 8), 0) // L
         == lax.broadcasted_iota(jnp.int32, (128, 8), 1)).astype(jnp.float32)
  dpp = jnp.dot(dp, sel, preferred_element_type=jnp.float32)
  dnp = jnp.dot(dn, sel, preferred_element_type=jnp.float32)
  lp = jnp.sum(dpp * dpp)
  h = jnp.maximum(0.0, MARGIN - jnp.sqrt(dnp))
  ln = jnp.sum(h * h)
  out_ref[0, 0] = (lp + ln) * (1000000.0 / (2.0 * (2 * B)))


def _half_mask(points):
  x = (points[:, 0] - 25) % X
  y = (points[:, 1] - 225) % Y
  z = (points[:, 2] - 28) % Z
  v = (x * Y + y) * Z + z
  return jnp.repeat((v & 1).astype(jnp.float32), L)


@jax.jit
def _impl(fix_image_feature, moving_image_feature, fixed_points,
          positive_points, negative_points):
  both = jnp.stack([fix_image_feature[0], moving_image_feature[0]])
  tab = both.transpose(0, 2, 3, 4, 1).reshape(XYZ, CP)
  dp, dn = _sc_gather_dist(
      fixed_points[:, 0], fixed_points[:, 1], fixed_points[:, 2],
      positive_points[:, 0], positive_points[:, 1], positive_points[:, 2],
      negative_points[:, 0], negative_points[:, 1], negative_points[:, 2],
      _half_mask(fixed_points), _half_mask(positive_points),
      _half_mask(negative_points),
      tab,
  )
  loss2d = pl.pallas_call(
      _loss_tc,
      out_shape=jax.ShapeDtypeStruct((1, 1), jnp.float32),
      in_specs=[pl.BlockSpec(memory_space=pltpu.VMEM)] * 2,
      out_specs=pl.BlockSpec(memory_space=pltpu.SMEM),
  )(dp.reshape(B * L // 128, 128), dn.reshape(B * L // 128, 128))
  return loss2d[0, 0]


def kernel(fix_image_feature, moving_image_feature, fixed_points,
           positive_points, negative_points):
  return _impl(fix_image_feature, moving_image_feature, fixed_points,
               positive_points, negative_points)


# speedup vs baseline: 1.5177x; 1.5177x over previous
import functools

import jax
import jax.numpy as jnp
from jax import lax
from jax.experimental import pallas as pl
from jax.experimental.pallas import tpu as pltpu
from jax.experimental.pallas import tpu_sc as plsc

MARGIN = 12.0
B = 4096
C = 64
X, Y, Z = 100, 88, 80
XYZ = X * Y * Z
CP = 128

NC, NS, L = 2, 16, 16
NW = NC * NS
PPW = B // NW
NCHUNK = PPW // L


def _pmod(a, n):
  r = lax.rem(a, n)
  return jnp.where(r < 0, r + n, r)


def _voxel_chunk(xv, yv, zv, j):
  s = pl.ds(j * L, L)
  xx = _pmod(xv[s] - 25, X)
  yy = _pmod(yv[s] - 225, Y)
  zz = _pmod(zv[s] - 28, Z)
  return (xx * Y + yy) * Z + zz


def _make_sc_kernel():
  mesh = plsc.VectorSubcoreMesh(core_axis_name="c", subcore_axis_name="s")

  @functools.partial(
      pl.kernel,
      mesh=mesh,
      out_type=(
          jax.ShapeDtypeStruct((B * L,), jnp.float32),
          jax.ShapeDtypeStruct((B * L,), jnp.float32),
      ),
      scratch_types=[
          pltpu.VMEM((PPW,), jnp.int32),
          pltpu.VMEM((PPW,), jnp.int32),
          pltpu.VMEM((PPW,), jnp.int32),
          pltpu.VMEM((PPW,), jnp.int32),
          pltpu.VMEM((PPW,), jnp.int32),
          pltpu.VMEM((PPW,), jnp.int32),
          pltpu.VMEM((PPW, CP), jnp.float32),
          pltpu.VMEM((PPW, CP), jnp.float32),
          pltpu.VMEM((PPW, CP), jnp.float32),
          pltpu.VMEM((PPW * L,), jnp.float32),
          pltpu.VMEM((PPW * L,), jnp.float32),
          pltpu.SemaphoreType.DMA,
      ],
  )
  def sc_kernel(fx_hbm, fy_hbm, fz_hbm, px_hbm, py_hbm, pz_hbm,
                nx_hbm, ny_hbm, nz_hbm, f_hbm, m_hbm, dp_hbm, dn_hbm,
                xv, yv, zv, idxf_v, idxp_v, idxn_v,
                rvf, rvp, rvn, ov, ov2, sem):
    wid = lax.axis_index("s") * NC + lax.axis_index("c")
    base = wid * PPW

    for cx, cy, cz, idx_v in ((fx_hbm, fy_hbm, fz_hbm, idxf_v),
                              (px_hbm, py_hbm, pz_hbm, idxp_v),
                              (nx_hbm, ny_hbm, nz_hbm, idxn_v)):
      pltpu.sync_copy(cx.at[pl.ds(base, PPW)], xv)
      pltpu.sync_copy(cy.at[pl.ds(base, PPW)], yv)
      pltpu.sync_copy(cz.at[pl.ds(base, PPW)], zv)
      for j in range(NCHUNK):
        idx_v[pl.ds(j * L, L)] = _voxel_chunk(xv, yv, zv, j)

    c1 = pltpu.async_copy(f_hbm.at[idxf_v], rvf, sem)
    c2 = pltpu.async_copy(m_hbm.at[idxp_v], rvp, sem)
    c3 = pltpu.async_copy(m_hbm.at[idxn_v], rvn, sem)
    c1.wait()
    c2.wait()
    c3.wait()

    def it(j, carry):
      for p in range(L):
        row = j * L + p
        accp = jnp.zeros((L,), jnp.float32)
        accn = jnp.zeros((L,), jnp.float32)
        for k in range(C // L):
          s = pl.ds(k * L, L)
          vf = rvf[row, s]
          vp = rvp[row, s]
          vn = rvn[row, s]
          accp = accp + (vf - vp) * (vf - vp)
          accn = accn + (vf - vn) * (vf - vn)
        ov[pl.ds(row * L, L)] = accp
        ov2[pl.ds(row * L, L)] = accn
      return carry

    lax.fori_loop(0, NCHUNK, it, 0)
    pltpu.sync_copy(ov, dp_hbm.at[pl.ds(base * L, PPW * L)])
    pltpu.sync_copy(ov2, dn_hbm.at[pl.ds(base * L, PPW * L)])

  return sc_kernel


_sc_gather_dist = _make_sc_kernel()


def _loss_tc(dp_ref, dn_ref, out_ref):
  dp = dp_ref[...]
  dn = dn_ref[...]
  sel = (lax.broadcasted_iota(jnp.int32, (128, 8), 0) // L
         == lax.broadcasted_iota(jnp.int32, (128, 8), 1)).astype(jnp.float32)
  dpp = jnp.dot(dp, sel, preferred_element_type=jnp.float32)
  dnp = jnp.dot(dn, sel, preferred_element_type=jnp.float32)
  lp = jnp.sum(dpp * dpp)
  h = jnp.maximum(0.0, MARGIN - jnp.sqrt(dnp))
  ln = jnp.sum(h * h)
  out_ref[0, 0] = (lp + ln) * (1000000.0 / (2.0 * (2 * B)))


@jax.jit
def _impl(fix_image_feature, moving_image_feature, fixed_points,
          positive_points, negative_points):
  pad = ((0, 0), (0, 0), (0, 0), (0, CP - C))
  ftab = jnp.pad(fix_image_feature[0].transpose(1, 2, 3, 0),
                 pad).reshape(XYZ, CP)
  mtab = jnp.pad(moving_image_feature[0].transpose(1, 2, 3, 0),
                 pad).reshape(XYZ, CP)
  dp, dn = _sc_gather_dist(
      fixed_points[:, 0], fixed_points[:, 1], fixed_points[:, 2],
      positive_points[:, 0], positive_points[:, 1], positive_points[:, 2],
      negative_points[:, 0], negative_points[:, 1], negative_points[:, 2],
      ftab, mtab,
  )
  loss2d = pl.pallas_call(
      _loss_tc,
      out_shape=jax.ShapeDtypeStruct((1, 1), jnp.float32),
      in_specs=[pl.BlockSpec(memory_space=pltpu.VMEM)] * 2,
      out_specs=pl.BlockSpec(memory_space=pltpu.SMEM),
  )(dp.reshape(B * L // 128, 128), dn.reshape(B * L // 128, 128))
  return loss2d[0, 0]


def kernel(fix_image_feature, moving_image_feature, fixed_points,
           positive_points, negative_points):
  return _impl(fix_image_feature, moving_image_feature, fixed_points,
               positive_points, negative_points)
